# baseline (speedup 1.0000x reference)
import functools

import jax
import jax.numpy as jnp
from jax import lax
from jax.experimental import pallas as pl
from jax.experimental.pallas import tpu as pltpu
from jax.experimental.pallas import tpu_sc as plsc

F = 768
H = 12
DH = 64
K = 16
S = 2048
B = 2

BM = 512
BA = 512

NC = 2
NS = 16
NW = NC * NS
QPW = B * S // NW

BAND = 1.0e6
INITK = 3.0e38


def _proj_kernel(x_ref, wq_ref, wk_ref, wv_ref, q_ref, k_ref, v_ref, ks_ref):
    i = pl.program_id(0)
    x = x_ref[...]
    q_ref[...] = jnp.dot(x, wq_ref[...], preferred_element_type=jnp.float32)
    kk = jnp.dot(x, wk_ref[...], preferred_element_type=jnp.float32)
    k_ref[...] = kk
    v_ref[...] = jnp.dot(x, wv_ref[...], preferred_element_type=jnp.float32)

    @pl.when(i % (S // BM) == 0)
    def _():
        ks_ref[...] = jnp.zeros_like(ks_ref)

    ks_ref[...] += jnp.sum(kk, axis=0, keepdims=True) * (1.0 / S)


_TAKE_DNUMS = lax.GatherDimensionNumbers(
    offset_dims=(), collapsed_slice_dims=(0,), start_index_map=(0,))


def _take(x, idx):
    return lax.gather(x, idx[:, None], _TAKE_DNUMS, slice_sizes=(1,),
                      mode=lax.GatherScatterMode.PROMISE_IN_BOUNDS)


def _bmax(x, iota):
    for s in (1, 2, 4, 8):
        x = jnp.maximum(x, _take(x, iota ^ s))
    return x


def _sc_knn_kernel(cx_hbm, cy_hbm, cz_hbm, lens_hbm, out_hbm,
                   cx_v, cy_v, cz_v, out_v, lens_v):
    wid = lax.axis_index("s") * NC + lax.axis_index("c")
    b = wid // NS
    iota = lax.iota(jnp.int32, 16)

    pltpu.sync_copy(cx_hbm.at[b], cx_v)
    pltpu.sync_copy(cy_hbm.at[b], cy_v)
    pltpu.sync_copy(cz_hbm.at[b], cz_v)
    pltpu.sync_copy(lens_hbm, lens_v)

    nvi = _bmax(jnp.where(iota == jnp.full((16,), b, jnp.int32),
                          lens_v[...], 0), iota)
    nv = nvi.astype(jnp.float32)
    nvi_max = jnp.maximum(nvi, 33)

    def insert_body(c):
        m, key, jf, bk0, bk1, bi0, bi1, mkv, miv = c
        fi = plsc.all_reduce_ffs(m)
        civ = _take(jf, fi)
        ckv = _take(key, fi)
        e0 = (bk0 == mkv) & (bi0 == miv)
        e1 = (bk1 == mkv) & (bi1 == miv)
        bk0 = jnp.where(e0, ckv, bk0)
        bi0 = jnp.where(e0, civ, bi0)
        bk1 = jnp.where(e1, ckv, bk1)
        bi1 = jnp.where(e1, civ, bi1)
        mkv = _bmax(jnp.maximum(bk0, bk1), iota)
        t0 = jnp.where(bk0 == mkv, bi0, -1.0)
        t1 = jnp.where(bk1 == mkv, bi1, -1.0)
        miv = _bmax(jnp.maximum(t0, t1), iota)
        m = m & (iota != fi) & (key < mkv)
        return m, key, jf, bk0, bk1, bi0, bi1, mkv, miv

    iota_f = iota.astype(jnp.float32)

    def valid_q(qq, q_local):
        qf = jnp.full((16,), q_local, jnp.int32)
        qx = plsc.load_gather(cx_v, [qf])
        qy = plsc.load_gather(cy_v, [qf])
        qz = plsc.load_gather(cz_v, [qf])

        def scan_cond(c):
            t = c[0]
            return jnp.any(jnp.full((16,), t * 64, jnp.int32) < nvi_max)

        def scan_body(c):
            t, bk0, bk1, bi0, bi1, mkv, miv = c
            keys = []
            jfs = []
            kmin = None
            for v in range(4):
                off = pl.multiple_of(t * 64 + v * 16, 16)
                dx = cx_v[pl.ds(off, 16)] - qx
                dy = cy_v[pl.ds(off, 16)] - qy
                dz = cz_v[pl.ds(off, 16)] - qz
                d2 = dx * dx + dy * dy + dz * dz
                jf = (iota + (t * 64 + v * 16)).astype(jnp.float32)
                key = jnp.where(jf < nv, d2, BAND + jf)
                keys.append(key)
                jfs.append(jf)
                kmin = key if kmin is None else jnp.minimum(kmin, key)

            def do_insert(k0, k1, k2, k3, j0, j1, j2, j3,
                          bk0, bk1, bi0, bi1, mkv, miv):
                for key, jf in ((k0, j0), (k1, j1), (k2, j2), (k3, j3)):
                    c2 = lax.while_loop(
                        lambda c: jnp.any(c[0]), insert_body,
                        (key < mkv, key, jf, bk0, bk1, bi0, bi1, mkv, miv))
                    bk0, bk1, bi0, bi1, mkv, miv = c2[3:]
                return bk0, bk1, bi0, bi1, mkv, miv

            bk0, bk1, bi0, bi1, mkv, miv = lax.cond(
                jnp.any(kmin < mkv), do_insert,
                lambda k0, k1, k2, k3, j0, j1, j2, j3,
                bk0, bk1, bi0, bi1, mkv, miv:
                (bk0, bk1, bi0, bi1, mkv, miv),
                keys[0], keys[1], keys[2], keys[3],
                jfs[0], jfs[1], jfs[2], jfs[3],
                bk0, bk1, bi0, bi1, mkv, miv)
            return t + 1, bk0, bk1, bi0, bi1, mkv, miv

        bk0 = jnp.full((16,), INITK, jnp.float32)
        bi0 = iota_f
        bk1 = jnp.full((16,), INITK, jnp.float32)
        bi1 = iota_f + 16.0
        mkv = jnp.full((16,), INITK, jnp.float32)
        miv = jnp.full((16,), 31.0, jnp.float32)
        _, bk0, bk1, bi0, bi1, mkv, miv = lax.while_loop(
            scan_cond, scan_body, (0, bk0, bk1, bi0, bi1, mkv, miv))

        base = qq * 64
        out_v[pl.ds(pl.multiple_of(base, 16), 16)] = bi0
        out_v[pl.ds(pl.multiple_of(base + 16, 16), 16)] = bi1
        out_v[pl.ds(pl.multiple_of(base + 32, 16), 16)] = bk0
        out_v[pl.ds(pl.multiple_of(base + 48, 16), 16)] = bk1
        return 0

    def invalid_q(qq, q_local):
        base = qq * 64
        out_v[pl.ds(pl.multiple_of(base, 16), 16)] = iota_f
        out_v[pl.ds(pl.multiple_of(base + 16, 16), 16)] = iota_f + 16.0
        out_v[pl.ds(pl.multiple_of(base + 32, 16), 16)] = iota_f
        out_v[pl.ds(pl.multiple_of(base + 48, 16), 16)] = iota_f + 16.0
        return 0

    def q_body(qq, carry):
        q_local = wid * QPW + qq - b * S
        valid = jnp.any(jnp.full((16,), q_local, jnp.int32) < nvi)
        return lax.cond(valid, valid_q, invalid_q, qq, q_local)

    lax.fori_loop(0, QPW, q_body, 0)
    pltpu.sync_copy(out_v, out_hbm.at[pl.ds(wid * (QPW * 64), QPW * 64)])


def _attn_kernel(lens_ref, q_ref, k_ref, v_ref, cand_ref, out_ref):
    b = pl.program_id(0)
    n = lens_ref[b]
    jidx = lax.broadcasted_iota(jnp.int32, (BA, S), 1)

    cand = cand_ref[...]
    idx = cand[:, :32]
    key = cand[:, 32:]
    key = jnp.where(key < 1.0e5, jnp.sqrt(key), key)

    lane = lax.broadcasted_iota(jnp.int32, (BA, 32), 1)
    rankm = jnp.zeros((BA, 32), jnp.int32)
    for t in range(32):
        kt = key[:, t:t + 1]
        it = idx[:, t:t + 1]
        less = jnp.logical_or(
            key < kt, jnp.logical_and(key == kt, idx < it))
        rt = jnp.sum(less.astype(jnp.int32), axis=1, keepdims=True)
        rankm = jnp.where(lane == t, rt, rankm)

    mask = jnp.zeros((BA, S), jnp.bool_)
    for r in range(1, K + 1):
        nbr = jnp.sum(jnp.where(rankm == r, idx, 0.0), axis=1, keepdims=True)
        mask = jnp.logical_or(mask, jidx == nbr.astype(jnp.int32))
    mask = jnp.logical_and(mask, jidx < n)

    scale = DH ** -0.5
    neg_inf = jnp.float32(-jnp.inf)
    for h in range(H):
        qh = q_ref[:, h * DH:(h + 1) * DH]
        kh = k_ref[:, h * DH:(h + 1) * DH]
        vh = v_ref[:, h * DH:(h + 1) * DH]
        logits = lax.dot_general(qh, kh, (((1,), (1,)), ((), ())),
                                 preferred_element_type=jnp.float32) * scale
        logits = jnp.where(mask, logits, neg_inf)
        rmax = jnp.max(logits, axis=1, keepdims=True)
        rmax = jnp.where(rmax > neg_inf, rmax, 0.0)
        p = jnp.where(mask, jnp.exp(logits - rmax), 0.0)
        denom = jnp.sum(p, axis=1, keepdims=True)
        p = p / jnp.where(denom > 0, denom, 1.0)
        out_ref[:, h * DH:(h + 1) * DH] = jnp.dot(
            p, vh, preferred_element_type=jnp.float32)


def kernel(x, coords, lens, Wq, Wk, Wv):
    x2d = x.reshape(B * S, F)
    lens = lens.astype(jnp.int32)

    q2d, k2d, v2d, ksum = pl.pallas_call(
        _proj_kernel,
        grid=(B * S // BM,),
        in_specs=[
            pl.BlockSpec((BM, F), lambda i: (i, 0)),
            pl.BlockSpec((F, F), lambda i: (0, 0)),
            pl.BlockSpec((F, F), lambda i: (0, 0)),
            pl.BlockSpec((F, F), lambda i: (0, 0)),
        ],
        out_specs=[
            pl.BlockSpec((BM, F), lambda i: (i, 0)),
            pl.BlockSpec((BM, F), lambda i: (i, 0)),
            pl.BlockSpec((BM, F), lambda i: (i, 0)),
            pl.BlockSpec((None, 1, F), lambda i: (i // (S // BM), 0, 0)),
        ],
        out_shape=[
            jax.ShapeDtypeStruct((B * S, F), jnp.float32),
            jax.ShapeDtypeStruct((B * S, F), jnp.float32),
            jax.ShapeDtypeStruct((B * S, F), jnp.float32),
            jax.ShapeDtypeStruct((B, 1, F), jnp.float32),
        ],
    )(x2d, Wq.T, Wk.T, Wv.T)

    cx = coords[:, :, 0]
    cy = coords[:, :, 1]
    cz = coords[:, :, 2]
    lens_pad = jnp.zeros((16,), jnp.int32).at[:B].set(lens)

    knn = pl.kernel(
        _sc_knn_kernel,
        mesh=plsc.VectorSubcoreMesh(core_axis_name="c", subcore_axis_name="s"),
        compiler_params=pltpu.CompilerParams(needs_layout_passes=False),
        out_type=jax.ShapeDtypeStruct((B * S * 64,), jnp.float32),
        scratch_types=[
            pltpu.VMEM((S,), jnp.float32),
            pltpu.VMEM((S,), jnp.float32),
            pltpu.VMEM((S,), jnp.float32),
            pltpu.VMEM((QPW * 64,), jnp.float32),
            pltpu.VMEM((16,), jnp.int32),
        ],
    )
    cand = knn(cx, cy, cz, lens_pad).reshape(B, S, 64)

    q3 = q2d.reshape(B, S, F)
    k3 = k2d.reshape(B, S, F)
    v3 = v2d.reshape(B, S, F)

    out = pl.pallas_call(
        _attn_kernel,
        grid=(B, S // BA),
        in_specs=[
            pl.BlockSpec(memory_space=pltpu.SMEM),
            pl.BlockSpec((None, BA, F), lambda b, i: (b, i, 0)),
            pl.BlockSpec((None, S, F), lambda b, i: (b, 0, 0)),
            pl.BlockSpec((None, S, F), lambda b, i: (b, 0, 0)),
            pl.BlockSpec((None, BA, 64), lambda b, i: (b, i, 0)),
        ],
        out_specs=pl.BlockSpec((None, BA, F), lambda b, i: (b, i, 0)),
        out_shape=jax.ShapeDtypeStruct((B, S, F), jnp.float32),
    )(lens, q3, k3, v3, cand)

    metric = ksum.reshape(B, H, DH)
    return (out, metric)


# speedup vs baseline: 1.2124x; 1.2124x over previous
import functools

import jax
import jax.numpy as jnp
from jax import lax
from jax.experimental import pallas as pl
from jax.experimental.pallas import tpu as pltpu
from jax.experimental.pallas import tpu_sc as plsc

F = 768
H = 12
DH = 64
K = 16
S = 2048
B = 2

BM = 512
BA = 512

NC = 2
NS = 16
NW = NC * NS
QPW = B * S // NW

BAND = 1.0e6
INITK = 3.0e38


def _proj_kernel(x_ref, wq_ref, wk_ref, wv_ref, q_ref, k_ref, v_ref, ks_ref):
    i = pl.program_id(0)
    x = x_ref[...]
    q_ref[...] = jnp.dot(x, wq_ref[...], preferred_element_type=jnp.float32)
    kk = jnp.dot(x, wk_ref[...], preferred_element_type=jnp.float32)
    k_ref[...] = kk
    v_ref[...] = jnp.dot(x, wv_ref[...], preferred_element_type=jnp.float32)

    @pl.when(i % (S // BM) == 0)
    def _():
        ks_ref[...] = jnp.zeros_like(ks_ref)

    ks_ref[...] += jnp.sum(kk, axis=0, keepdims=True) * (1.0 / S)


_TAKE_DNUMS = lax.GatherDimensionNumbers(
    offset_dims=(), collapsed_slice_dims=(0,), start_index_map=(0,))


def _take(x, idx):
    return lax.gather(x, idx[:, None], _TAKE_DNUMS, slice_sizes=(1,),
                      mode=lax.GatherScatterMode.PROMISE_IN_BOUNDS)


def _bmax(x, iota):
    for s in (1, 2, 4, 8):
        x = jnp.maximum(x, _take(x, iota ^ s))
    return x


def _sc_knn_kernel(cx_hbm, cy_hbm, cz_hbm, lens_hbm, out_hbm,
                   cx_v, cy_v, cz_v, out_v, lens_v):
    wid = lax.axis_index("s") * NC + lax.axis_index("c")
    b = wid // NS
    iota = lax.iota(jnp.int32, 16)

    pltpu.sync_copy(cx_hbm.at[b], cx_v)
    pltpu.sync_copy(cy_hbm.at[b], cy_v)
    pltpu.sync_copy(cz_hbm.at[b], cz_v)
    pltpu.sync_copy(lens_hbm, lens_v)

    nvi = _bmax(jnp.where(iota == jnp.full((16,), b, jnp.int32),
                          lens_v[...], 0), iota)
    nv = nvi.astype(jnp.float32)
    nvi_max = jnp.maximum(nvi, 33)

    iota_f = iota.astype(jnp.float32)
    fifteen = jnp.full((16,), 15, jnp.int32)

    def valid_q(qq, q_local):
        qf = jnp.full((16,), q_local, jnp.int32)
        qx = plsc.load_gather(cx_v, [qf])
        qy = plsc.load_gather(cy_v, [qf])
        qz = plsc.load_gather(cz_v, [qf])

        def scan_cond(c):
            t = c[0]
            return jnp.any(jnp.full((16,), t * 64, jnp.int32) < nvi_max)

        def merge16(key, jf, bk0, bk1, bi0, bi1):
            ck, ci = plsc.sort_key_val(key, jf)
            crk = lax.rev(ck, (0,))
            cri = lax.rev(ci, (0,))
            sel = bk1 <= crk
            n1k = jnp.where(sel, bk1, crk)
            n1i = jnp.where(sel, bi1, cri)
            n1k, n1i = plsc.sort_key_val(n1k, n1i)
            r1k = lax.rev(n1k, (0,))
            r1i = lax.rev(n1i, (0,))
            sel2 = bk0 <= r1k
            lok = jnp.where(sel2, bk0, r1k)
            loi = jnp.where(sel2, bi0, r1i)
            hik = jnp.where(sel2, r1k, bk0)
            hii = jnp.where(sel2, r1i, bi0)
            lok, loi = plsc.sort_key_val(lok, loi)
            hik, hii = plsc.sort_key_val(hik, hii)
            return lok, hik, loi, hii

        def scan_body(c):
            t, bk0, bk1, bi0, bi1, thr = c
            keys = []
            jfs = []
            kmin = None
            for v in range(4):
                off = pl.multiple_of(t * 64 + v * 16, 16)
                dx = cx_v[pl.ds(off, 16)] - qx
                dy = cy_v[pl.ds(off, 16)] - qy
                dz = cz_v[pl.ds(off, 16)] - qz
                d2 = dx * dx + dy * dy + dz * dz
                jf = (iota + (t * 64 + v * 16)).astype(jnp.float32)
                key = jnp.where(jf < nv, d2, BAND + jf)
                keys.append(key)
                jfs.append(jf)
                kmin = key if kmin is None else jnp.minimum(kmin, key)

            def do_insert(k0, k1, k2, k3, j0, j1, j2, j3,
                          bk0, bk1, bi0, bi1, thr):
                for key, jf in ((k0, j0), (k1, j1), (k2, j2), (k3, j3)):
                    bk0, bk1, bi0, bi1 = lax.cond(
                        jnp.any(key < thr), merge16,
                        lambda key, jf, bk0, bk1, bi0, bi1:
                        (bk0, bk1, bi0, bi1),
                        key, jf, bk0, bk1, bi0, bi1)
                    thr = _take(bk1, fifteen)
                return bk0, bk1, bi0, bi1, thr

            bk0, bk1, bi0, bi1, thr = lax.cond(
                jnp.any(kmin < thr), do_insert,
                lambda k0, k1, k2, k3, j0, j1, j2, j3,
                bk0, bk1, bi0, bi1, thr:
                (bk0, bk1, bi0, bi1, thr),
                keys[0], keys[1], keys[2], keys[3],
                jfs[0], jfs[1], jfs[2], jfs[3],
                bk0, bk1, bi0, bi1, thr)
            return t + 1, bk0, bk1, bi0, bi1, thr

        bk0 = jnp.full((16,), INITK, jnp.float32)
        bi0 = iota_f
        bk1 = jnp.full((16,), INITK, jnp.float32)
        bi1 = iota_f + 16.0
        thr = jnp.full((16,), INITK, jnp.float32)
        _, bk0, bk1, bi0, bi1, thr = lax.while_loop(
            scan_cond, scan_body, (0, bk0, bk1, bi0, bi1, thr))

        base = qq * 64
        out_v[pl.ds(pl.multiple_of(base, 16), 16)] = bi0
        out_v[pl.ds(pl.multiple_of(base + 16, 16), 16)] = bi1
        out_v[pl.ds(pl.multiple_of(base + 32, 16), 16)] = bk0
        out_v[pl.ds(pl.multiple_of(base + 48, 16), 16)] = bk1
        return 0

    def invalid_q(qq, q_local):
        base = qq * 64
        out_v[pl.ds(pl.multiple_of(base, 16), 16)] = iota_f
        out_v[pl.ds(pl.multiple_of(base + 16, 16), 16)] = iota_f + 16.0
        out_v[pl.ds(pl.multiple_of(base + 32, 16), 16)] = iota_f
        out_v[pl.ds(pl.multiple_of(base + 48, 16), 16)] = iota_f + 16.0
        return 0

    def q_body(qq, carry):
        q_local = wid * QPW + qq - b * S
        valid = jnp.any(jnp.full((16,), q_local, jnp.int32) < nvi)
        return lax.cond(valid, valid_q, invalid_q, qq, q_local)

    lax.fori_loop(0, QPW, q_body, 0)
    pltpu.sync_copy(out_v, out_hbm.at[pl.ds(wid * (QPW * 64), QPW * 64)])


def _attn_kernel(lens_ref, q_ref, k_ref, v_ref, cand_ref, out_ref):
    b = pl.program_id(0)
    n = lens_ref[b]
    jidx = lax.broadcasted_iota(jnp.int32, (BA, S), 1)

    cand = cand_ref[...]
    idx = cand[:, :32]
    key = cand[:, 32:]
    key = jnp.where(key < 1.0e5, jnp.sqrt(key), key)

    lane = lax.broadcasted_iota(jnp.int32, (BA, 32), 1)
    rankm = jnp.zeros((BA, 32), jnp.int32)
    for t in range(32):
        kt = key[:, t:t + 1]
        it = idx[:, t:t + 1]
        less = jnp.logical_or(
            key < kt, jnp.logical_and(key == kt, idx < it))
        rt = jnp.sum(less.astype(jnp.int32), axis=1, keepdims=True)
        rankm = jnp.where(lane == t, rt, rankm)

    mask = jnp.zeros((BA, S), jnp.bool_)
    for r in range(1, K + 1):
        nbr = jnp.sum(jnp.where(rankm == r, idx, 0.0), axis=1, keepdims=True)
        mask = jnp.logical_or(mask, jidx == nbr.astype(jnp.int32))
    mask = jnp.logical_and(mask, jidx < n)

    scale = DH ** -0.5
    neg_inf = jnp.float32(-jnp.inf)
    for h in range(H):
        qh = q_ref[:, h * DH:(h + 1) * DH]
        kh = k_ref[:, h * DH:(h + 1) * DH]
        vh = v_ref[:, h * DH:(h + 1) * DH]
        logits = lax.dot_general(qh, kh, (((1,), (1,)), ((), ())),
                                 preferred_element_type=jnp.float32) * scale
        logits = jnp.where(mask, logits, neg_inf)
        rmax = jnp.max(logits, axis=1, keepdims=True)
        rmax = jnp.where(rmax > neg_inf, rmax, 0.0)
        p = jnp.where(mask, jnp.exp(logits - rmax), 0.0)
        denom = jnp.sum(p, axis=1, keepdims=True)
        p = p / jnp.where(denom > 0, denom, 1.0)
        out_ref[:, h * DH:(h + 1) * DH] = jnp.dot(
            p, vh, preferred_element_type=jnp.float32)


def kernel(x, coords, lens, Wq, Wk, Wv):
    x2d = x.reshape(B * S, F)
    lens = lens.astype(jnp.int32)

    q2d, k2d, v2d, ksum = pl.pallas_call(
        _proj_kernel,
        grid=(B * S // BM,),
        in_specs=[
            pl.BlockSpec((BM, F), lambda i: (i, 0)),
            pl.BlockSpec((F, F), lambda i: (0, 0)),
            pl.BlockSpec((F, F), lambda i: (0, 0)),
            pl.BlockSpec((F, F), lambda i: (0, 0)),
        ],
        out_specs=[
            pl.BlockSpec((BM, F), lambda i: (i, 0)),
            pl.BlockSpec((BM, F), lambda i: (i, 0)),
            pl.BlockSpec((BM, F), lambda i: (i, 0)),
            pl.BlockSpec((None, 1, F), lambda i: (i // (S // BM), 0, 0)),
        ],
        out_shape=[
            jax.ShapeDtypeStruct((B * S, F), jnp.float32),
            jax.ShapeDtypeStruct((B * S, F), jnp.float32),
            jax.ShapeDtypeStruct((B * S, F), jnp.float32),
            jax.ShapeDtypeStruct((B, 1, F), jnp.float32),
        ],
    )(x2d, Wq.T, Wk.T, Wv.T)

    cx = coords[:, :, 0]
    cy = coords[:, :, 1]
    cz = coords[:, :, 2]
    lens_pad = jnp.zeros((16,), jnp.int32).at[:B].set(lens)

    knn = pl.kernel(
        _sc_knn_kernel,
        mesh=plsc.VectorSubcoreMesh(core_axis_name="c", subcore_axis_name="s"),
        compiler_params=pltpu.CompilerParams(needs_layout_passes=False),
        out_type=jax.ShapeDtypeStruct((B * S * 64,), jnp.float32),
        scratch_types=[
            pltpu.VMEM((S,), jnp.float32),
            pltpu.VMEM((S,), jnp.float32),
            pltpu.VMEM((S,), jnp.float32),
            pltpu.VMEM((QPW * 64,), jnp.float32),
            pltpu.VMEM((16,), jnp.int32),
        ],
    )
    cand = knn(cx, cy, cz, lens_pad).reshape(B, S, 64)

    q3 = q2d.reshape(B, S, F)
    k3 = k2d.reshape(B, S, F)
    v3 = v2d.reshape(B, S, F)

    out = pl.pallas_call(
        _attn_kernel,
        grid=(B, S // BA),
        in_specs=[
            pl.BlockSpec(memory_space=pltpu.SMEM),
            pl.BlockSpec((None, BA, F), lambda b, i: (b, i, 0)),
            pl.BlockSpec((None, S, F), lambda b, i: (b, 0, 0)),
            pl.BlockSpec((None, S, F), lambda b, i: (b, 0, 0)),
            pl.BlockSpec((None, BA, 64), lambda b, i: (b, i, 0)),
        ],
        out_specs=pl.BlockSpec((None, BA, F), lambda b, i: (b, i, 0)),
        out_shape=jax.ShapeDtypeStruct((B, S, F), jnp.float32),
    )(lens, q3, k3, v3, cand)

    metric = ksum.reshape(B, H, DH)
    return (out, metric)


# speedup vs baseline: 1.2188x; 1.0053x over previous
import jax
import jax.numpy as jnp
from jax import lax
from jax.experimental import pallas as pl
from jax.experimental.pallas import tpu as pltpu
from jax.experimental.pallas import tpu_sc as plsc

F = 768
H = 12
DH = 64
K = 16
S = 2048
B = 2

BM = 512
BA = 512

NC = 2
NS = 16
NW = NC * NS
QPW = B * S // NW

BAND = 1.0e6
INITK = 3.0e38


def _proj_kernel(x_ref, wq_ref, wk_ref, wv_ref, q_ref, k_ref, v_ref, ks_ref):
    i = pl.program_id(0)
    x = x_ref[...]
    q_ref[...] = jnp.dot(x, wq_ref[...], preferred_element_type=jnp.float32)
    kk = jnp.dot(x, wk_ref[...], preferred_element_type=jnp.float32)
    k_ref[...] = kk
    v_ref[...] = jnp.dot(x, wv_ref[...], preferred_element_type=jnp.float32)

    @pl.when(i % (S // BM) == 0)
    def _():
        ks_ref[...] = jnp.zeros_like(ks_ref)

    ks_ref[...] += jnp.sum(kk, axis=0, keepdims=True) * (1.0 / S)


_TAKE_DNUMS = lax.GatherDimensionNumbers(
    offset_dims=(), collapsed_slice_dims=(0,), start_index_map=(0,))


def _take(x, idx):
    return lax.gather(x, idx[:, None], _TAKE_DNUMS, slice_sizes=(1,),
                      mode=lax.GatherScatterMode.PROMISE_IN_BOUNDS)


def _bmax(x, iota):
    for s in (1, 2, 4, 8):
        x = jnp.maximum(x, _take(x, iota ^ s))
    return x


def _sc_knn_kernel(cx_hbm, cy_hbm, cz_hbm, lens_hbm, out_hbm,
                   cx_v, cy_v, cz_v, out_v, lens_v):
    wid = lax.axis_index("s") * NC + lax.axis_index("c")
    b = wid // NS
    iota = lax.iota(jnp.int32, 16)

    pltpu.sync_copy(cx_hbm.at[b], cx_v)
    pltpu.sync_copy(cy_hbm.at[b], cy_v)
    pltpu.sync_copy(cz_hbm.at[b], cz_v)
    pltpu.sync_copy(lens_hbm, lens_v)

    nvi = _bmax(jnp.where(iota == jnp.full((16,), b, jnp.int32),
                          lens_v[...], 0), iota)
    nv = nvi.astype(jnp.float32)
    nvi_max = jnp.maximum(nvi, 33)

    iota_f = iota.astype(jnp.float32)
    fifteen = jnp.full((16,), 15, jnp.int32)

    def valid_q(qq, q_local):
        qf = jnp.full((16,), q_local, jnp.int32)
        qx = plsc.load_gather(cx_v, [qf])
        qy = plsc.load_gather(cy_v, [qf])
        qz = plsc.load_gather(cz_v, [qf])

        def scan_cond(c):
            t = c[0]
            return jnp.any(jnp.full((16,), t * 128, jnp.int32) < nvi_max)

        def merge16(key, jf, bk0, bk1, bi0, bi1):
            ck, ci = plsc.sort_key_val(key, jf)
            crk = lax.rev(ck, (0,))
            cri = lax.rev(ci, (0,))
            sel = bk1 <= crk
            n1k = jnp.where(sel, bk1, crk)
            n1i = jnp.where(sel, bi1, cri)
            n1k, n1i = plsc.sort_key_val(n1k, n1i)
            r1k = lax.rev(n1k, (0,))
            r1i = lax.rev(n1i, (0,))
            sel2 = bk0 <= r1k
            lok = jnp.where(sel2, bk0, r1k)
            loi = jnp.where(sel2, bi0, r1i)
            hik = jnp.where(sel2, r1k, bk0)
            hii = jnp.where(sel2, r1i, bi0)
            lok, loi = plsc.sort_key_val(lok, loi)
            hik, hii = plsc.sort_key_val(hik, hii)
            return lok, hik, loi, hii

        def scan_body(c):
            t, bk0, bk1, bi0, bi1, thr = c
            keys = []
            jfs = []
            kmin = None
            for v in range(8):
                off = pl.multiple_of(t * 128 + v * 16, 16)
                dx = cx_v[pl.ds(off, 16)] - qx
                dy = cy_v[pl.ds(off, 16)] - qy
                dz = cz_v[pl.ds(off, 16)] - qz
                d2 = dx * dx + dy * dy + dz * dz
                jf = (iota + (t * 128 + v * 16)).astype(jnp.float32)
                key = jnp.where(jf < nv, d2, BAND + jf)
                keys.append(key)
                jfs.append(jf)
                kmin = key if kmin is None else jnp.minimum(kmin, key)

            def do_insert(*args):
                ks = args[:8]
                js = args[8:16]
                bk0, bk1, bi0, bi1, thr = args[16:]
                for key, jf in zip(ks, js):
                    bk0, bk1, bi0, bi1 = lax.cond(
                        jnp.any(key < thr), merge16,
                        lambda key, jf, bk0, bk1, bi0, bi1:
                        (bk0, bk1, bi0, bi1),
                        key, jf, bk0, bk1, bi0, bi1)
                    thr = _take(bk1, fifteen)
                return bk0, bk1, bi0, bi1, thr

            def no_insert(*args):
                return args[16:]

            bk0, bk1, bi0, bi1, thr = lax.cond(
                jnp.any(kmin < thr), do_insert, no_insert,
                *keys, *jfs, bk0, bk1, bi0, bi1, thr)
            return t + 1, bk0, bk1, bi0, bi1, thr

        bk0 = jnp.full((16,), INITK, jnp.float32)
        bi0 = iota_f
        bk1 = jnp.full((16,), INITK, jnp.float32)
        bi1 = iota_f + 16.0
        thr = jnp.full((16,), INITK, jnp.float32)
        _, bk0, bk1, bi0, bi1, thr = lax.while_loop(
            scan_cond, scan_body, (0, bk0, bk1, bi0, bi1, thr))

        base = qq * 64
        out_v[pl.ds(pl.multiple_of(base, 16), 16)] = bi0
        out_v[pl.ds(pl.multiple_of(base + 16, 16), 16)] = bi1
        out_v[pl.ds(pl.multiple_of(base + 32, 16), 16)] = bk0
        out_v[pl.ds(pl.multiple_of(base + 48, 16), 16)] = bk1
        return 0

    def invalid_q(qq, q_local):
        base = qq * 64
        out_v[pl.ds(pl.multiple_of(base, 16), 16)] = iota_f
        out_v[pl.ds(pl.multiple_of(base + 16, 16), 16)] = iota_f + 16.0
        out_v[pl.ds(pl.multiple_of(base + 32, 16), 16)] = iota_f
        out_v[pl.ds(pl.multiple_of(base + 48, 16), 16)] = iota_f + 16.0
        return 0

    def q_body(qq, carry):
        q_local = wid * QPW + qq - b * S
        valid = jnp.any(jnp.full((16,), q_local, jnp.int32) < nvi)
        return lax.cond(valid, valid_q, invalid_q, qq, q_local)

    lax.fori_loop(0, QPW, q_body, 0)
    pltpu.sync_copy(out_v, out_hbm.at[pl.ds(wid * (QPW * 64), QPW * 64)])


def _attn_kernel(lens_ref, q_ref, k_ref, v_ref, cand_ref, out_ref):
    b = pl.program_id(0)
    n = lens_ref[b]
    jidx = lax.broadcasted_iota(jnp.int32, (BA, S), 1)

    cand = cand_ref[...]
    idx = cand[:, :32]
    key = cand[:, 32:]
    key = jnp.where(key < 1.0e5, jnp.sqrt(key), key)

    lane = lax.broadcasted_iota(jnp.int32, (BA, 32), 1)
    rankm = jnp.zeros((BA, 32), jnp.int32)
    for t in range(32):
        kt = key[:, t:t + 1]
        it = idx[:, t:t + 1]
        less = jnp.logical_or(
            key < kt, jnp.logical_and(key == kt, idx < it))
        rt = jnp.sum(less.astype(jnp.int32), axis=1, keepdims=True)
        rankm = jnp.where(lane == t, rt, rankm)

    mask = jnp.zeros((BA, S), jnp.bool_)
    for r in range(1, K + 1):
        nbr = jnp.sum(jnp.where(rankm == r, idx, 0.0), axis=1, keepdims=True)
        mask = jnp.logical_or(mask, jidx == nbr.astype(jnp.int32))
    mask = jnp.logical_and(mask, jidx < n)

    scale = DH ** -0.5
    neg_inf = jnp.float32(-jnp.inf)
    for h in range(H):
        qh = q_ref[:, h * DH:(h + 1) * DH]
        kh = k_ref[:, h * DH:(h + 1) * DH]
        vh = v_ref[:, h * DH:(h + 1) * DH]
        logits = lax.dot_general(qh, kh, (((1,), (1,)), ((), ())),
                                 preferred_element_type=jnp.float32) * scale
        logits = jnp.where(mask, logits, neg_inf)
        rmax = jnp.max(logits, axis=1, keepdims=True)
        rmax = jnp.where(rmax > neg_inf, rmax, 0.0)
        p = jnp.where(mask, jnp.exp(logits - rmax), 0.0)
        denom = jnp.sum(p, axis=1, keepdims=True)
        p = p / jnp.where(denom > 0, denom, 1.0)
        out_ref[:, h * DH:(h + 1) * DH] = jnp.dot(
            p, vh, preferred_element_type=jnp.float32)


def kernel(x, coords, lens, Wq, Wk, Wv):
    x2d = x.reshape(B * S, F)
    lens = lens.astype(jnp.int32)

    q2d, k2d, v2d, ksum = pl.pallas_call(
        _proj_kernel,
        grid=(B * S // BM,),
        in_specs=[
            pl.BlockSpec((BM, F), lambda i: (i, 0)),
            pl.BlockSpec((F, F), lambda i: (0, 0)),
            pl.BlockSpec((F, F), lambda i: (0, 0)),
            pl.BlockSpec((F, F), lambda i: (0, 0)),
        ],
        out_specs=[
            pl.BlockSpec((BM, F), lambda i: (i, 0)),
            pl.BlockSpec((BM, F), lambda i: (i, 0)),
            pl.BlockSpec((BM, F), lambda i: (i, 0)),
            pl.BlockSpec((None, 1, F), lambda i: (i // (S // BM), 0, 0)),
        ],
        out_shape=[
            jax.ShapeDtypeStruct((B * S, F), jnp.float32),
            jax.ShapeDtypeStruct((B * S, F), jnp.float32),
            jax.ShapeDtypeStruct((B * S, F), jnp.float32),
            jax.ShapeDtypeStruct((B, 1, F), jnp.float32),
        ],
    )(x2d, Wq.T, Wk.T, Wv.T)

    cx = coords[:, :, 0]
    cy = coords[:, :, 1]
    cz = coords[:, :, 2]
    lens_pad = jnp.zeros((16,), jnp.int32).at[:B].set(lens)

    knn = pl.kernel(
        _sc_knn_kernel,
        mesh=plsc.VectorSubcoreMesh(core_axis_name="c", subcore_axis_name="s"),
        compiler_params=pltpu.CompilerParams(needs_layout_passes=False),
        out_type=jax.ShapeDtypeStruct((B * S * 64,), jnp.float32),
        scratch_types=[
            pltpu.VMEM((S,), jnp.float32),
            pltpu.VMEM((S,), jnp.float32),
            pltpu.VMEM((S,), jnp.float32),
            pltpu.VMEM((QPW * 64,), jnp.float32),
            pltpu.VMEM((16,), jnp.int32),
        ],
    )
    cand = knn(cx, cy, cz, lens_pad).reshape(B, S, 64)

    q3 = q2d.reshape(B, S, F)
    k3 = k2d.reshape(B, S, F)
    v3 = v2d.reshape(B, S, F)

    out = pl.pallas_call(
        _attn_kernel,
        grid=(B, S // BA),
        in_specs=[
            pl.BlockSpec(memory_space=pltpu.SMEM),
            pl.BlockSpec((None, BA, F), lambda b, i: (b, i, 0)),
            pl.BlockSpec((None, S, F), lambda b, i: (b, 0, 0)),
            pl.BlockSpec((None, S, F), lambda b, i: (b, 0, 0)),
            pl.BlockSpec((None, BA, 64), lambda b, i: (b, i, 0)),
        ],
        out_specs=pl.BlockSpec((None, BA, F), lambda b, i: (b, i, 0)),
        out_shape=jax.ShapeDtypeStruct((B, S, F), jnp.float32),
    )(lens, q3, k3, v3, cand)

    metric = ksum.reshape(B, H, DH)
    return (out, metric)
